# baseline (speedup 1.0000x reference)
import functools

import jax
import jax.numpy as jnp
from jax import lax
from jax.experimental import pallas as pl
from jax.experimental.pallas import tpu as pltpu
from jax.experimental.pallas import tpu_sc as plsc

_L = 16
_ROWS = 64
_COLS = 8192
_CHUNKS = _COLS // _L
_NW = 32
_ROWS_PER_W = _ROWS // _NW
_UNROLL = 8
_NBINS = 256

_IMIN = -(2 ** 31)
_IMAXP = 2 ** 31 - 1


def _chunk_loop(body, carry, n_chunks=_CHUNKS, unroll=_UNROLL):
    def outer(i, c):
        for u in range(unroll):
            c = body(i * (unroll * _L) + u * _L, c)
        return c
    return lax.fori_loop(0, n_chunks // unroll, outer, carry)


def _splat(x, lane):
    idx = jnp.broadcast_to(jnp.int32(lane), (_L,))
    return jnp.take(x, idx, mode="wrap")


def _tec_body(tok_hbm, kv_hbm, out_hbm, row_v, key_v, cbuf_v, hist_v, kv_v):
    wid = lax.axis_index("s") * 2 + lax.axis_index("c")

    pltpu.sync_copy(kv_hbm, kv_v)
    kvec = kv_v[...]
    krv = kvec - 1

    zeros = jnp.zeros((_L,), jnp.int32)
    ones = zeros + 1
    iota = lax.iota(jnp.int32, _L)
    lane0 = iota == 0
    delta = plsc.cumsum(ones) - iota

    for r in range(_ROWS_PER_W):
        row = wid * _ROWS_PER_W + r
        pltpu.sync_copy(tok_hbm.at[row], row_v)

        def zero_body(base, c):
            hist_v[pl.ds(base, _L)] = zeros
            return c
        _chunk_loop(zero_body, zeros, n_chunks=_NBINS * _L // _L)

        def key_body(base, c):
            x = row_v[pl.ds(base, _L)]
            b = lax.bitcast_convert_type(x, jnp.int32) ^ _IMIN
            ks = jnp.where(b < 0, b ^ _IMAXP, b)
            key_v[pl.ds(base, _L)] = ks
            d = lax.shift_right_logical(ks ^ _IMIN, 24)
            plsc.addupdate_scatter(hist_v, [iota * _NBINS + d], ones)
            return c
        _chunk_loop(key_body, zeros)

        k0 = key_v[pl.ds(0, _L)]
        dnat = lax.shift_right_logical(k0 ^ _IMIN, 24)
        plsc.addupdate_scatter(hist_v, [iota * _NBINS + dnat], -ones, mask=lane0)
        plsc.addupdate_scatter(hist_v, [iota * _NBINS], ones, mask=lane0)
        key_v[pl.ds(0, _L)] = jnp.where(lane0, _IMIN, k0)

        def scan_body(t, st):
            cumv, d0v, belowv = st
            tot = hist_v[pl.ds(0 * _NBINS + t * _L, _L)]
            for l in range(1, _L):
                tot = tot + hist_v[pl.ds(l * _NBINS + t * _L, _L)]
            pc = plsc.cumsum(tot) - tot * (delta - 1)
            pc = pc + cumv
            fnd = pc > krv
            ffs = plsc.all_reduce_ffs(fnd)
            has = ffs < _L
            pick = has & (d0v >= _NBINS)
            prev = jnp.take(pc, jnp.maximum(ffs - 1, 0),
                            mode="wrap")
            below_cand = jnp.where(ffs == 0, cumv, prev)
            d0v = jnp.where(pick, t * _L + ffs, d0v)
            belowv = jnp.where(pick, below_cand, belowv)
            cumv = _splat(pc, _L - 1)
            return cumv, d0v, belowv
        _, d0v, belowv = lax.fori_loop(
            0, _NBINS // _L, scan_body, (zeros, zeros + _NBINS, zeros))

        def comp_body(base, pos):
            ks = key_v[pl.ds(base, _L)]
            d = lax.shift_right_logical(ks ^ _IMIN, 24)
            m = d == d0v
            plsc.store_compressed(cbuf_v.at[pl.ds(pos, _L)], ks, mask=m)
            return pos + jnp.sum(jnp.where(m, 1, 0))
        pos = _chunk_loop(comp_body, jnp.int32(0))
        cbuf_v[pl.ds(pos, _L)] = zeros + _IMAXP
        nch = lax.shift_right_logical(pos + (_L - 1), 4)

        candtop = lax.shift_left(d0v ^ (_NBINS // 2), 24)
        krg = krv - belowv

        def bit_body(_, st):
            puv, bitv = st
            candlow = puv | bitv
            candv = candtop | candlow
            def cnt_body(j, cnt):
                m = cbuf_v[pl.ds(j * _L, _L)] < candv
                return cnt + plsc.all_reduce_population_count(m)
            cnt = lax.fori_loop(0, nch, cnt_body, zeros)
            take_ = cnt <= krg
            return jnp.where(take_, candlow, puv), lax.shift_right_logical(bitv, ones)
        puv, _ = lax.fori_loop(0, 24, bit_body, (zeros, zeros + (1 << 23)))
        t_key = candtop | puv

        def clg_body(j, cnt):
            m = cbuf_v[pl.ds(j * _L, _L)] < t_key
            return cnt + plsc.all_reduce_population_count(m)
        count_less = belowv + lax.fori_loop(0, nch, clg_body, zeros)
        quota = kvec - count_less

        def mask_body(base, carry):
            c = key_v[pl.ds(base, _L)]
            ltm = c < t_key
            eqm = c == t_key
            eqi = jnp.where(eqm, 1, 0)
            excl = plsc.cumsum(eqi) - eqi * delta + carry
            keep = ltm | (eqm & (excl < quota))
            key_v[pl.ds(base, _L)] = jnp.where(keep, 1, 0)
            return carry + plsc.all_reduce_population_count(eqm)
        _chunk_loop(mask_body, zeros)

        pltpu.sync_copy(key_v, out_hbm.at[row])


@jax.jit
def _select_mask(token_sequence, kvec):
    mesh = plsc.VectorSubcoreMesh(core_axis_name="c", subcore_axis_name="s")
    f = pl.kernel(
        _tec_body,
        out_type=jax.ShapeDtypeStruct((_ROWS, _COLS), jnp.int32),
        mesh=mesh,
        scratch_types=[
            pltpu.VMEM((_COLS,), jnp.float32),
            pltpu.VMEM((_COLS,), jnp.int32),
            pltpu.VMEM((_COLS + _L,), jnp.int32),
            pltpu.VMEM((_NBINS * _L,), jnp.int32),
            pltpu.VMEM((_L,), jnp.int32),
        ],
        compiler_params=pltpu.CompilerParams(needs_layout_passes=False),
    )
    return f(token_sequence, kvec)


def kernel(token_sequence, embedding_sequence, compression_rate):
    seq_len = token_sequence.shape[1]
    c = compression_rate.reshape(-1)[0]
    scaled = seq_len * c
    fs = jnp.floor(scaled)
    k = jnp.where(scaled == fs, seq_len - fs, seq_len - fs - 1.0).astype(jnp.int32)
    k = jnp.maximum(k, 1)
    kvec = jnp.broadcast_to(k, (_L,)).astype(jnp.int32)
    mask = _select_mask(token_sequence, kvec)
    y = mask.astype(bool)
    return (y, y)


# speedup vs baseline: 1.2537x; 1.2537x over previous
import functools

import jax
import jax.numpy as jnp
from jax import lax
from jax.experimental import pallas as pl
from jax.experimental.pallas import tpu as pltpu
from jax.experimental.pallas import tpu_sc as plsc

_L = 16
_ROWS = 64
_COLS = 8192
_CHUNKS = _COLS // _L
_NW = 32
_ROWS_PER_W = _ROWS // _NW
_UNROLL = 8
_NBINS = 256
_LEVELS = 4

_IMIN = -(2 ** 31)
_IMAXP = 2 ** 31 - 1


def _chunk_loop(body, carry, n_chunks=_CHUNKS, unroll=_UNROLL):
    def outer(i, c):
        for u in range(unroll):
            c = body(i * (unroll * _L) + u * _L, c)
        return c
    return lax.fori_loop(0, n_chunks // unroll, outer, carry)


def _splat(x, lane):
    idx = jnp.broadcast_to(jnp.int32(lane), (_L,))
    return jnp.take(x, idx, mode="wrap")


def _tec_body(tok_hbm, kv_hbm, out_hbm, row_v, key_v, hist_v, kv_v):
    wid = lax.axis_index("s") * 2 + lax.axis_index("c")

    pltpu.sync_copy(kv_hbm, kv_v)
    kvec = kv_v[...]
    krv = kvec - 1

    zeros = jnp.zeros((_L,), jnp.int32)
    ones = zeros + 1
    iota = lax.iota(jnp.int32, _L)
    lane0 = iota == 0
    delta = plsc.cumsum(ones) - iota

    def hist_scan(krg):
        def scan_body(t, st):
            cumv, d0v, belowv = st
            tot = hist_v[pl.ds(t * _L, _L)]
            for l in range(1, _L):
                tot = tot + hist_v[pl.ds(l * _NBINS + t * _L, _L)]
            pc = plsc.cumsum(tot) - tot * (delta - 1)
            pc = pc + cumv
            fnd = pc > krg
            ffs = plsc.all_reduce_ffs(fnd)
            pick = (ffs < _L) & (d0v >= _NBINS)
            prev = jnp.take(pc, jnp.maximum(ffs - 1, 0), mode="wrap")
            below_cand = jnp.where(ffs == 0, cumv, prev)
            d0v = jnp.where(pick, t * _L + ffs, d0v)
            belowv = jnp.where(pick, below_cand, belowv)
            cumv = _splat(pc, _L - 1)
            return cumv, d0v, belowv
        _, d0v, belowv = lax.fori_loop(
            0, _NBINS // _L, scan_body, (zeros, zeros + _NBINS, zeros))
        return d0v, belowv

    def zero_hist():
        def zero_body(base, c):
            hist_v[pl.ds(base, _L)] = zeros
            return c
        _chunk_loop(zero_body, zeros, n_chunks=_NBINS)

    for r in range(_ROWS_PER_W):
        row = wid * _ROWS_PER_W + r
        pltpu.sync_copy(tok_hbm.at[row], row_v)

        below_tot = zeros
        prefv = zeros
        for lvl in range(_LEVELS):
            zero_hist()
            dsh = 24 - 8 * lvl
            if lvl == 0:
                def h_body(base, c, _dsh=dsh):
                    x = row_v[pl.ds(base, _L)]
                    b = lax.bitcast_convert_type(x, jnp.int32) ^ _IMIN
                    ks = jnp.where(b < 0, b ^ _IMAXP, b)
                    key_v[pl.ds(base, _L)] = ks
                    d = lax.shift_right_logical(ks ^ _IMIN, _dsh)
                    plsc.addupdate_scatter(hist_v, [iota * _NBINS + d], ones)
                    return c
                _chunk_loop(h_body, zeros)
                k0 = key_v[pl.ds(0, _L)]
                dnat = lax.shift_right_logical(k0 ^ _IMIN, dsh)
                plsc.addupdate_scatter(hist_v, [iota * _NBINS + dnat], -ones,
                                       mask=lane0)
                plsc.addupdate_scatter(hist_v, [iota * _NBINS], ones, mask=lane0)
                key_v[pl.ds(0, _L)] = jnp.where(lane0, _IMIN, k0)
            else:
                psh = 32 - 8 * lvl
                def h_body(base, c, _dsh=dsh, _psh=psh, _pref=prefv):
                    ku = key_v[pl.ds(base, _L)] ^ _IMIN
                    m = lax.shift_right_logical(ku, _psh) == _pref
                    d = lax.shift_right_logical(ku, _dsh) & 255
                    plsc.addupdate_scatter(hist_v, [iota * _NBINS + d], ones,
                                           mask=m)
                    return c
                _chunk_loop(h_body, zeros)
            d0v, belowv = hist_scan(krv - below_tot)
            below_tot = below_tot + belowv
            prefv = lax.shift_left(prefv, 8) | d0v

        t_key = prefv ^ _IMIN
        quota = kvec - below_tot

        def mask_body(base, carry):
            c = key_v[pl.ds(base, _L)]
            ltm = c < t_key
            eqm = c == t_key
            eqi = jnp.where(eqm, 1, 0)
            excl = plsc.cumsum(eqi) - eqi * delta + carry
            keep = ltm | (eqm & (excl < quota))
            key_v[pl.ds(base, _L)] = jnp.where(keep, 1, 0)
            return carry + plsc.all_reduce_population_count(eqm)
        _chunk_loop(mask_body, zeros)

        pltpu.sync_copy(key_v, out_hbm.at[row])


@jax.jit
def _select_mask(token_sequence, kvec):
    mesh = plsc.VectorSubcoreMesh(core_axis_name="c", subcore_axis_name="s")
    f = pl.kernel(
        _tec_body,
        out_type=jax.ShapeDtypeStruct((_ROWS, _COLS), jnp.int32),
        mesh=mesh,
        scratch_types=[
            pltpu.VMEM((_COLS,), jnp.float32),
            pltpu.VMEM((_COLS,), jnp.int32),
            pltpu.VMEM((_NBINS * _L,), jnp.int32),
            pltpu.VMEM((_L,), jnp.int32),
        ],
        compiler_params=pltpu.CompilerParams(needs_layout_passes=False),
    )
    return f(token_sequence, kvec)


def kernel(token_sequence, embedding_sequence, compression_rate):
    seq_len = token_sequence.shape[1]
    c = compression_rate.reshape(-1)[0]
    scaled = seq_len * c
    fs = jnp.floor(scaled)
    k = jnp.where(scaled == fs, seq_len - fs, seq_len - fs - 1.0).astype(jnp.int32)
    k = jnp.maximum(k, 1)
    kvec = jnp.broadcast_to(k, (_L,)).astype(jnp.int32)
    mask = _select_mask(token_sequence, kvec)
    y = mask.astype(bool)
    return (y, y)
